# baseline (speedup 1.0000x reference)
import jax
import jax.numpy as jnp
from jax import lax
from jax.experimental import pallas as pl
from jax.experimental.pallas import tpu as pltpu

_G = 8
_SWEEPS = 10
_SKIP_EPS = 1.1920929e-08


def _diag(n):
    row = lax.broadcasted_iota(jnp.int32, (n, n), 0)
    col = lax.broadcasted_iota(jnp.int32, (n, n), 1)
    return row == col


def _encode_kernel(lr_ref, w0_ref, a_ref, gc_ref):
    f32 = jnp.float32
    bf16 = jnp.bfloat16
    lr = lr_ref[...]
    g, n, _ = lr.shape
    h = w0_ref.shape[1]
    r_col = lax.rsqrt(jnp.sum(lr, axis=2, keepdims=True))
    r_col = jnp.where(jnp.isinf(r_col), 0.0, r_col)
    r_row = lax.rsqrt(jnp.sum(lr, axis=1, keepdims=True))
    r_row = jnp.where(jnp.isinf(r_row), 0.0, r_row)
    a = (r_col * lr) * r_row
    a_ref[...] = a
    a_stack = a.astype(bf16).reshape(g * n, n)
    gc = jnp.dot(a_stack, w0_ref[...].astype(bf16),
                 preferred_element_type=f32)
    gc_ref[...] = jnp.maximum(gc, 0.0).astype(bf16).reshape(g, n, h)


def _rot_params(app, aqq, apq):
    f32 = jnp.float32
    two_apq = f32(2.0) * apq
    tau = (aqq - app) * pl.reciprocal(two_apq, approx=True)
    xx = f32(1.0) + tau * tau
    sq = lax.rsqrt(xx) * xx
    sq = jnp.where(xx == jnp.inf, xx, sq)
    sq = jnp.where(tau >= f32(0.0), sq, -sq)
    t = pl.reciprocal(sq + tau, approx=True)
    skip = jnp.abs(apq) <= f32(_SKIP_EPS) * jnp.minimum(jnp.abs(app),
                                                        jnp.abs(aqq))
    t = jnp.where(skip, f32(0.0), t)
    c = lax.rsqrt(f32(1.0) + t * t)
    s = c * t
    return c, s, t


def _jacobi_kernel(a_ref, v_ref, w_ref):
    f32 = jnp.float32
    g, n, _ = a_ref.shape
    m = n // 2
    eye_m = _diag(m)
    eye_n = _diag(n)
    a0 = a_ref[...]
    v0 = jnp.broadcast_to(jnp.where(eye_n, f32(1.0), f32(0.0)), (g, n, n))

    def body(_, carry):
        a, v = carry
        tl = a[:, :m, :m]
        tr = a[:, :m, m:]
        br = a[:, m:, m:]
        zq = f32(0.0)
        app_c = jnp.sum(jnp.where(eye_m, tl, zq), axis=2, keepdims=True)
        aqq_c = jnp.sum(jnp.where(eye_m, br, zq), axis=2, keepdims=True)
        apq_c = jnp.sum(jnp.where(eye_m, tr, zq), axis=2, keepdims=True)
        c_c, s_c, t_c = _rot_params(app_c, aqq_c, apq_c)
        app_r = jnp.sum(jnp.where(eye_m, tl, zq), axis=1, keepdims=True)
        aqq_r = jnp.sum(jnp.where(eye_m, br, zq), axis=1, keepdims=True)
        apq_r = jnp.sum(jnp.where(eye_m, tr, zq), axis=1, keepdims=True)
        c_r, s_r, _ = _rot_params(app_r, aqq_r, apq_r)
        t_half = a[:, :m, :]
        b_half = a[:, m:, :]
        tn = c_c * t_half - s_c * b_half
        bn = s_c * t_half + c_c * b_half
        a1 = jnp.concatenate([tn, bn], axis=1)
        lc = a1[:, :, :m]
        rc = a1[:, :, m:]
        ln = lc * c_r - rc * s_r
        rn = lc * s_r + rc * c_r
        tapq = t_c * apq_c
        app2 = app_c - tapq
        aqq2 = tapq + aqq_c
        tl2 = jnp.where(eye_m, app2, ln[:, :m])
        bl2 = jnp.where(eye_m, zq, ln[:, m:])
        tr2 = jnp.where(eye_m, zq, rn[:, :m])
        br2 = jnp.where(eye_m, aqq2, rn[:, m:])
        top = jnp.concatenate([tl2, tr2], axis=2)
        bot = jnp.concatenate([bl2, br2], axis=2)
        tp = jnp.concatenate([top[:, 0:1], bot[:, 0:1], top[:, 1:m - 1]], axis=1)
        bp = jnp.concatenate([bot[:, 1:m], top[:, m - 1:m]], axis=1)
        a2 = jnp.concatenate([tp, bp], axis=1)
        lp = a2[:, :, :m]
        rp = a2[:, :, m:]
        lq = jnp.concatenate([lp[..., 0:1], rp[..., 0:1], lp[..., 1:m - 1]],
                             axis=2)
        rq = jnp.concatenate([rp[..., 1:m], lp[..., m - 1:m]], axis=2)
        a3 = jnp.concatenate([lq, rq], axis=2)
        vl = v[:, :, :m]
        vr = v[:, :, m:]
        vln = vl * c_r - vr * s_r
        vrn = vl * s_r + vr * c_r
        vlp = jnp.concatenate([vln[..., 0:1], vrn[..., 0:1], vln[..., 1:m - 1]],
                              axis=2)
        vrp = jnp.concatenate([vrn[..., 1:m], vln[..., m - 1:m]], axis=2)
        v2 = jnp.concatenate([vlp, vrp], axis=2)
        return a3, v2

    a_fin, v_fin = lax.fori_loop(0, _SWEEPS * (n - 1), body, (a0, v0))
    v_ref[...] = v_fin
    w_ref[...] = jnp.sum(jnp.where(eye_n, a_fin, 0.0), axis=2)


def _decoder_kernel(aw_ref, u_ref, x_ref, w1_ref, w2_ref, z_ref):
    f32 = jnp.float32
    bf16 = jnp.bfloat16
    aw = aw_ref[...].astype(bf16)
    w1 = w1_ref[...].astype(bf16)
    w2 = w2_ref[...].astype(bf16)
    g, n, _ = u_ref.shape
    h = aw.shape[0]
    diag = _diag(h)
    u_stack = u_ref[...].astype(bf16).reshape(g * n, n)
    dn_bt = (((1,), (1,)), ((), ()))
    b_all = lax.dot_general(aw, u_stack, dn_bt,
                            preferred_element_type=f32)
    for i in range(g):
        b = b_all[:, i * n:(i + 1) * n].astype(bf16)
        x = x_ref[i]
        f_d = jnp.abs(jnp.dot(b, x, preferred_element_type=f32))
        adj = jnp.where(diag, 1.0, f_d).astype(bf16)
        xo = lax.dot_general(adj, adj, dn_bt, preferred_element_type=f32)
        z0 = jnp.abs(jnp.where(diag, 1.0, xo)).astype(bf16)
        h1 = jnp.dot(z0, w1, preferred_element_type=f32).astype(bf16)
        h2 = jnp.dot(adj, h1, preferred_element_type=f32)
        h2 = jnp.maximum(h2, 0.0).astype(bf16)
        o1 = jnp.dot(h2, w2, preferred_element_type=f32).astype(bf16)
        o2 = jnp.tanh(jnp.dot(adj, o1, preferred_element_type=f32))
        out = (o2 + o2.T) * 0.5
        z_ref[i] = jnp.where(diag, 0.0, out)


def kernel(lr_batch, gc0_w, gsr_w, gc1_w, gc2_w):
    f32 = jnp.float32
    lr_batch = lr_batch.astype(f32)
    batch, n, _ = lr_batch.shape
    h = gc0_w.shape[1]
    hid = gc1_w.shape[1]
    g = _G if batch % _G == 0 else 1
    steps = batch // g
    par = pltpu.CompilerParams(dimension_semantics=("parallel",))

    a_mat, gc = pl.pallas_call(
        _encode_kernel,
        grid=(steps,),
        out_shape=(jax.ShapeDtypeStruct((batch, n, n), f32),
                   jax.ShapeDtypeStruct((batch, n, h), jnp.bfloat16)),
        in_specs=[pl.BlockSpec((g, n, n), lambda s: (s, 0, 0)),
                  pl.BlockSpec((n, h), lambda s: (0, 0))],
        out_specs=(pl.BlockSpec((g, n, n), lambda s: (s, 0, 0)),
                   pl.BlockSpec((g, n, h), lambda s: (s, 0, 0))),
        compiler_params=par,
    )(lr_batch, gc0_w)

    a_w = gsr_w[:, :n] + gsr_w[:, n:]

    a_sym = (a_mat + jnp.swapaxes(a_mat, -1, -2)) * f32(0.5)
    v_mat, w_vals = pl.pallas_call(
        _jacobi_kernel,
        grid=(steps,),
        out_shape=(jax.ShapeDtypeStruct((batch, n, n), f32),
                   jax.ShapeDtypeStruct((batch, n), f32)),
        in_specs=[pl.BlockSpec((g, n, n), lambda s: (s, 0, 0))],
        out_specs=(pl.BlockSpec((g, n, n), lambda s: (s, 0, 0)),
                   pl.BlockSpec((g, n), lambda s: (s, 0))),
        compiler_params=par,
    )(a_sym)
    order = jnp.argsort(w_vals, axis=-1, stable=True)
    u_mat = jnp.take_along_axis(v_mat, order[:, None, :], axis=2)

    z = pl.pallas_call(
        _decoder_kernel,
        grid=(steps,),
        out_shape=jax.ShapeDtypeStruct((batch, h, h), f32),
        in_specs=[pl.BlockSpec((h, n), lambda s: (0, 0)),
                  pl.BlockSpec((g, n, n), lambda s: (s, 0, 0)),
                  pl.BlockSpec((g, n, h), lambda s: (s, 0, 0)),
                  pl.BlockSpec((h, hid), lambda s: (0, 0)),
                  pl.BlockSpec((hid, h), lambda s: (0, 0))],
        out_specs=pl.BlockSpec((g, h, h), lambda s: (s, 0, 0)),
        compiler_params=par,
    )(a_w, u_mat, gc, gc1_w, gc2_w)
    return z


# speedup vs baseline: 1.3992x; 1.3992x over previous
import jax
import jax.numpy as jnp
from jax import lax
from jax.experimental import pallas as pl
from jax.experimental.pallas import tpu as pltpu

_G = 8


def _diag(n):
    row = lax.broadcasted_iota(jnp.int32, (n, n), 0)
    col = lax.broadcasted_iota(jnp.int32, (n, n), 1)
    return row == col


def _encode_kernel(lr_ref, w0_ref, a_ref, gc_ref):
    f32 = jnp.float32
    bf16 = jnp.bfloat16
    lr = lr_ref[...]
    g, n, _ = lr.shape
    h = w0_ref.shape[1]
    r_col = lax.rsqrt(jnp.sum(lr, axis=2, keepdims=True))
    r_col = jnp.where(jnp.isinf(r_col), 0.0, r_col)
    r_row = lax.rsqrt(jnp.sum(lr, axis=1, keepdims=True))
    r_row = jnp.where(jnp.isinf(r_row), 0.0, r_row)
    a = (r_col * lr) * r_row
    a_ref[...] = a
    a_stack = a.astype(bf16).reshape(g * n, n)
    gc = jnp.dot(a_stack, w0_ref[...].astype(bf16),
                 preferred_element_type=f32)
    gc_ref[...] = jnp.maximum(gc, 0.0).astype(bf16).reshape(g, n, h)


def _decoder_kernel(aw_ref, u_ref, x_ref, w1_ref, w2_ref, z_ref):
    f32 = jnp.float32
    bf16 = jnp.bfloat16
    aw = aw_ref[...].astype(bf16)
    w1 = w1_ref[...].astype(bf16)
    w2 = w2_ref[...].astype(bf16)
    g, n, _ = u_ref.shape
    h = aw.shape[0]
    diag = _diag(h)
    u_stack = u_ref[...].astype(bf16).reshape(g * n, n)
    dn_bt = (((1,), (1,)), ((), ()))
    b_all = lax.dot_general(aw, u_stack, dn_bt,
                            preferred_element_type=f32)
    for i in range(g):
        b = b_all[:, i * n:(i + 1) * n].astype(bf16)
        x = x_ref[i]
        f_d = jnp.abs(jnp.dot(b, x, preferred_element_type=f32))
        adj = jnp.where(diag, 1.0, f_d).astype(bf16)
        xo = lax.dot_general(adj, adj, dn_bt, preferred_element_type=f32)
        z0 = jnp.abs(jnp.where(diag, 1.0, xo)).astype(bf16)
        h1 = jnp.dot(z0, w1, preferred_element_type=f32).astype(bf16)
        h2 = jnp.dot(adj, h1, preferred_element_type=f32)
        h2 = jnp.maximum(h2, 0.0).astype(bf16)
        o1 = jnp.dot(h2, w2, preferred_element_type=f32).astype(bf16)
        o2 = jnp.tanh(jnp.dot(adj, o1, preferred_element_type=f32))
        out = (o2 + o2.T) * 0.5
        z_ref[i] = jnp.where(diag, 0.0, out)


def kernel(lr_batch, gc0_w, gsr_w, gc1_w, gc2_w):
    f32 = jnp.float32
    lr_batch = lr_batch.astype(f32)
    batch, n, _ = lr_batch.shape
    h = gc0_w.shape[1]
    hid = gc1_w.shape[1]
    g = _G if batch % _G == 0 else 1
    steps = batch // g
    par = pltpu.CompilerParams(dimension_semantics=("parallel",))

    a_mat, gc = pl.pallas_call(
        _encode_kernel,
        grid=(steps,),
        out_shape=(jax.ShapeDtypeStruct((batch, n, n), f32),
                   jax.ShapeDtypeStruct((batch, n, h), jnp.bfloat16)),
        in_specs=[pl.BlockSpec((g, n, n), lambda s: (s, 0, 0)),
                  pl.BlockSpec((n, h), lambda s: (0, 0))],
        out_specs=(pl.BlockSpec((g, n, n), lambda s: (s, 0, 0)),
                   pl.BlockSpec((g, n, h), lambda s: (s, 0, 0))),
        compiler_params=par,
    )(lr_batch, gc0_w)

    a_w = gsr_w[:, :n] + gsr_w[:, n:]
    _, u_mat = jnp.linalg.eigh(a_mat, UPLO="U")
    u_mat = u_mat.astype(f32)

    z = pl.pallas_call(
        _decoder_kernel,
        grid=(steps,),
        out_shape=jax.ShapeDtypeStruct((batch, h, h), f32),
        in_specs=[pl.BlockSpec((h, n), lambda s: (0, 0)),
                  pl.BlockSpec((g, n, n), lambda s: (s, 0, 0)),
                  pl.BlockSpec((g, n, h), lambda s: (s, 0, 0)),
                  pl.BlockSpec((h, hid), lambda s: (0, 0)),
                  pl.BlockSpec((hid, h), lambda s: (0, 0))],
        out_specs=pl.BlockSpec((g, h, h), lambda s: (s, 0, 0)),
        compiler_params=par,
    )(a_w, u_mat, gc, gc1_w, gc2_w)
    return z
